# baseline (speedup 1.0000x reference)
import functools

import jax
import jax.numpy as jnp
from jax import lax
from jax.experimental import pallas as pl
from jax.experimental.pallas import tpu as pltpu
from jax.experimental.pallas import tpu_sc as plsc

N_NODES_C = 10000
N_EDGES_C = 320000
B_GRAPHS_C = 100
NPG_C = 100
FG_EDGES_C = 1600
HID_C = 128
K_C = 32
N_LAYERS_C = 8

EDGE_BLK = 2560
NODE_BLK = 2000

SC_NC = 2
SC_NS = 16
SC_W = SC_NC * SC_NS
K1_EPW = N_EDGES_C // SC_W
K1_BATCH = 200
K1_ITERS = K1_EPW // K1_BATCH
K1_ROUNDS = (K1_ITERS + 1) // 2


def _sc_mesh():
    return plsc.VectorSubcoreMesh(core_axis_name="c", subcore_axis_name="s",
                                  num_cores=SC_NC, num_subcores=SC_NS)


def _k1_body(src_hbm, dst_hbm, xi_hbm, xj_hbm, g_hbm,
             srcv0, srcv1, dstv0, dstv1, a0, a1, b0, b1, sg0, sg1):
    wid = lax.axis_index("s") * SC_NC + lax.axis_index("c")
    srcv = (srcv0, srcv1)
    dstv = (dstv0, dstv1)
    abuf = (a0, a1)
    bbuf = (b0, b1)
    sg = (sg0, sg1)

    def issue(slot, it):
        base = wid * K1_EPW + it * K1_BATCH
        pltpu.sync_copy(src_hbm.at[pl.ds(base, K1_BATCH)], srcv[slot])
        pltpu.sync_copy(dst_hbm.at[pl.ds(base, K1_BATCH)], dstv[slot])
        pltpu.async_copy(xi_hbm.at[srcv[slot]], abuf[slot], sg[slot])
        pltpu.async_copy(xj_hbm.at[dstv[slot]], bbuf[slot], sg[slot])

    issue(0, 0)
    issue(1, 1)

    def round_body(r, carry):
        for slot in (0, 1):
            it = r * 2 + slot
            pltpu.make_async_copy(xi_hbm.at[srcv[slot]], abuf[slot],
                                  sg[slot]).wait()
            pltpu.make_async_copy(xj_hbm.at[dstv[slot]], bbuf[slot],
                                  sg[slot]).wait()

            def row(rr, c2, slot=slot):
                for j in range(HID_C // 16):
                    sl = pl.ds(j * 16, 16)
                    abuf[slot][rr, sl] = abuf[slot][rr, sl] + bbuf[slot][rr, sl]
                return c2

            lax.fori_loop(0, K1_BATCH, row, 0)
            base = wid * K1_EPW + it * K1_BATCH
            pltpu.sync_copy(abuf[slot], g_hbm.at[pl.ds(base, K1_BATCH)])

            @pl.when(it + 2 < K1_ITERS)
            def _(slot=slot, it=it):
                issue(slot, it + 2)
        return carry

    lax.fori_loop(0, K1_ROUNDS, round_body, 0)


def _edge_gather_add(src, dst, xi, xj):
    return pl.kernel(
        _k1_body,
        out_type=jax.ShapeDtypeStruct((N_EDGES_C, HID_C), jnp.float32),
        mesh=_sc_mesh(),
        scratch_types=[
            pltpu.VMEM((K1_BATCH,), jnp.int32),
            pltpu.VMEM((K1_BATCH,), jnp.int32),
            pltpu.VMEM((K1_BATCH,), jnp.int32),
            pltpu.VMEM((K1_BATCH,), jnp.int32),
            pltpu.VMEM((K1_BATCH, HID_C), jnp.float32),
            pltpu.VMEM((K1_BATCH, HID_C), jnp.float32),
            pltpu.VMEM((K1_BATCH, HID_C), jnp.float32),
            pltpu.VMEM((K1_BATCH, HID_C), jnp.float32),
            pltpu.SemaphoreType.DMA,
            pltpu.SemaphoreType.DMA,
        ],
    )(src, dst, xi, xj)


def _fwd_body(ha_ref, hb_ref, da_ref, db_ref, w_ref, b_ref, o_ref):
    x = _norm_x(ha_ref, hb_ref, da_ref, db_ref)
    y = jnp.dot(x, w_ref[...], preferred_element_type=jnp.float32) + b_ref[...]
    o_ref[...] = jnp.maximum(y, 0.0)


def _fwd_matmul(ha, hb, da, db, w, b):
    n, _ = ha.shape
    grid = (n // NODE_BLK,)
    hspec = pl.BlockSpec((NODE_BLK, HID_C), lambda i: (i, 0))
    dspec = pl.BlockSpec((1, 1, NODE_BLK), lambda i: (i, 0, 0))
    return pl.pallas_call(
        _fwd_body,
        grid=grid,
        in_specs=[
            hspec, hspec, dspec, dspec,
            pl.BlockSpec((HID_C, HID_C), lambda i: (0, 0)),
            pl.BlockSpec((1, HID_C), lambda i: (0, 0)),
        ],
        out_specs=pl.BlockSpec((NODE_BLK, HID_C), lambda i: (i, 0)),
        out_shape=jax.ShapeDtypeStruct((n, HID_C), jnp.float32),
    )(ha, hb, da, db, w, b)


def _norm_x(ha_ref, hb_ref, da_ref, db_ref):
    d = (da_ref[...] + db_ref[...]).reshape(NODE_BLK)
    s = jnp.where(d > 0.0, 1.0 / d, 0.0)[:, None]
    return jnp.maximum((ha_ref[...] + hb_ref[...]) * s, 0.0)


def _nodeN_body1(nout, h_ref, w_ref, *o_refs):
    out = jnp.dot(h_ref[...], w_ref[...], preferred_element_type=jnp.float32)
    for k in range(nout):
        o_refs[k][...] = out[:, k * HID_C:(k + 1) * HID_C]


def _nodeN_body2(nout, ha_ref, hb_ref, da_ref, db_ref, w_ref, *o_refs):
    x = _norm_x(ha_ref, hb_ref, da_ref, db_ref)
    out = jnp.dot(x, w_ref[...], preferred_element_type=jnp.float32)
    for k in range(nout):
        o_refs[k][...] = out[:, k * HID_C:(k + 1) * HID_C]


def _node_matmul(hs, w_cat, nout):
    n = hs[0].shape[0]
    grid = (n // NODE_BLK,)
    hspec = pl.BlockSpec((NODE_BLK, HID_C), lambda i: (i, 0))
    dspec = pl.BlockSpec((1, 1, NODE_BLK), lambda i: (i, 0, 0))
    ospec = pl.BlockSpec((NODE_BLK, HID_C), lambda i: (i, 0))
    oshape = jax.ShapeDtypeStruct((n, HID_C), jnp.float32)
    if len(hs) == 1:
        body = functools.partial(_nodeN_body1, nout)
        specs = [hspec]
    else:
        body = functools.partial(_nodeN_body2, nout)
        specs = [hspec, hspec, dspec, dspec]
    out = pl.pallas_call(
        body,
        grid=grid,
        in_specs=specs + [pl.BlockSpec((HID_C, nout * HID_C), lambda i: (0, 0))],
        out_specs=[ospec] * nout,
        out_shape=[oshape] * nout,
    )(*hs, w_cat)
    return out


def _ef_body(e_ref, w_ref, b_ref, o_ref):
    o_ref[...] = jnp.dot(e_ref[...], w_ref[...],
                         preferred_element_type=jnp.float32) + b_ref[...]


def _ef_kernel(e, w, b):
    grid = (N_EDGES_C // EDGE_BLK,)
    return pl.pallas_call(
        _ef_body,
        grid=grid,
        in_specs=[
            pl.BlockSpec((EDGE_BLK, HID_C), lambda i: (i, 0)),
            pl.BlockSpec((HID_C, HID_C), lambda i: (0, 0)),
            pl.BlockSpec((1, HID_C), lambda i: (0, 0)),
        ],
        out_specs=pl.BlockSpec((EDGE_BLK, HID_C), lambda i: (i, 0)),
        out_shape=jax.ShapeDtypeStruct((N_EDGES_C, HID_C), jnp.float32),
    )(e, w, b)


def _edge_body(ef_ref, g_ref, a_ref, fo_ref, lo_ref, bm_ref):
    f = ef_ref[...] + g_ref[...]
    fo = jnp.where(f >= 0.0, f, 0.2 * f)
    fo_ref[...] = fo
    lo = fo * a_ref[...]
    lo = jnp.sum(lo, axis=1)
    lo_ref[...] = lo.reshape(1, 1, EDGE_BLK)
    bm_ref[...] = jnp.max(lo.reshape(EDGE_BLK // HID_C, HID_C), axis=0).reshape(1, 1, HID_C)


def _edge_kernel(ef, g, attn):
    grid = (N_EDGES_C // EDGE_BLK,)
    return pl.pallas_call(
        _edge_body,
        grid=grid,
        in_specs=[
            pl.BlockSpec((EDGE_BLK, HID_C), lambda i: (i, 0)),
            pl.BlockSpec((EDGE_BLK, HID_C), lambda i: (i, 0)),
            pl.BlockSpec((1, HID_C), lambda i: (0, 0)),
        ],
        out_specs=[
            pl.BlockSpec((EDGE_BLK, HID_C), lambda i: (i, 0)),
            pl.BlockSpec((1, 1, EDGE_BLK), lambda i: (i, 0, 0)),
            pl.BlockSpec((1, 1, HID_C), lambda i: (i, 0, 0)),
        ],
        out_shape=[
            jax.ShapeDtypeStruct((N_EDGES_C, HID_C), jnp.float32),
            jax.ShapeDtypeStruct((N_EDGES_C // EDGE_BLK, 1, EDGE_BLK), jnp.float32),
            jax.ShapeDtypeStruct((N_EDGES_C // EDGE_BLK, 1, HID_C), jnp.float32),
        ],
    )(ef, g, attn)


EC_C = N_EDGES_C // EDGE_BLK
ND_C = 10
NBLK_C = N_NODES_C // ND_C
DEN_SUB = 256


def _ex_body(lo_ref, bm_ref, ex_ref):
    big_l = jnp.max(bm_ref[...])
    ex_ref[...] = jnp.exp(lo_ref[...] - big_l)


def _ex_kernel(lo3d, bmax):
    return pl.pallas_call(
        _ex_body,
        grid=(EC_C,),
        in_specs=[
            pl.BlockSpec((1, 1, EDGE_BLK), lambda i: (i, 0, 0)),
            pl.BlockSpec((EC_C, 1, HID_C), lambda i: (0, 0, 0)),
        ],
        out_specs=pl.BlockSpec((1, 1, EDGE_BLK), lambda i: (i, 0, 0)),
        out_shape=jax.ShapeDtypeStruct((EC_C, 1, EDGE_BLK), jnp.float32),
    )(lo3d, bmax)


K3_BATCH = 80
K3_ITERS = K1_EPW // K3_BATCH
K3_ROUNDS = (K3_ITERS + 1) // 2


def _k3_body(src_hbm, dst_hbm, ex_hbm, xn_hbm, hpart_hbm, hden_hbm,
             srcv0, srcv1, dstv0, dstv1, exv0, exv1, r0, r1,
             zbuf, zdbuf, sg0, sg1, shared, shden):
    cid = lax.axis_index("c")
    sid = lax.axis_index("s")
    wid = sid * SC_NC + cid
    srcv = (srcv0, srcv1)
    dstv = (dstv0, dstv1)
    exv = (exv0, exv1)
    rows = (r0, r1)
    sg = (sg0, sg1)

    def zero16(r, carry):
        for j in range(HID_C // 16):
            zbuf[r, pl.ds(j * 16, 16)] = jnp.zeros((16,), jnp.float32)
        return carry

    lax.fori_loop(0, zbuf.shape[0], zero16, 0)

    def zero1d(q, carry):
        zdbuf[pl.ds(q * 16, 16)] = jnp.zeros((16,), jnp.float32)
        return carry

    lax.fori_loop(0, 2000 // 16, zero1d, 0)

    @pl.when(sid < 5)
    def _():
        for k in range(20):
            pltpu.sync_copy(zbuf, shared.at[pl.ds(sid * 2000 + k * 100, 100)])
        pltpu.sync_copy(zdbuf, shden.at[pl.ds(sid * 2000, 2000)])

    plsc.subcore_barrier()

    def issue(slot, it):
        base = wid * K1_EPW + it * K3_BATCH
        pltpu.sync_copy(src_hbm.at[pl.ds(base, K3_BATCH)], srcv[slot])
        pltpu.sync_copy(dst_hbm.at[pl.ds(base, K3_BATCH)], dstv[slot])
        pltpu.sync_copy(ex_hbm.at[pl.ds(base, K3_BATCH)], exv[slot])
        pltpu.async_copy(xn_hbm.at[srcv[slot]], rows[slot], sg[slot])

    issue(0, 0)
    issue(1, 1)

    def round_body(r, carry):
        for slot in (0, 1):
            it = r * 2 + slot

            @pl.when(it < K3_ITERS)
            def _(slot=slot, it=it):
                pltpu.make_async_copy(xn_hbm.at[srcv[slot]], rows[slot],
                                      sg[slot]).wait()

                def scale_group(q, c2, slot=slot):
                    a16 = exv[slot][pl.ds(q * 16, 16)]
                    for l in range(16):
                        s = a16[l]
                        rr = q * 16 + l
                        for j in range(HID_C // 16):
                            sl2 = pl.ds(j * 16, 16)
                            rows[slot][rr, sl2] = rows[slot][rr, sl2] * s
                    return c2

                lax.fori_loop(0, K3_BATCH // 16, scale_group, 0)
                pltpu.sync_copy(rows[slot], shared.at[dstv[slot]], add=True)
                pltpu.sync_copy(exv[slot], shden.at[dstv[slot]], add=True)

                @pl.when(it + 2 < K3_ITERS)
                def _(slot=slot, it=it):
                    issue(slot, it + 2)
        return carry

    lax.fori_loop(0, K3_ROUNDS, round_body, 0)
    plsc.subcore_barrier()

    @pl.when(sid < 5)
    def _():
        pltpu.sync_copy(
            shared.at[pl.ds(sid * 2000, 2000)],
            hpart_hbm.at[cid, pl.ds(sid * 2000, 2000)])
        pltpu.sync_copy(shden.at[pl.ds(sid * 2000, 2000)], zdbuf)
        pltpu.sync_copy(
            zdbuf, hden_hbm.at[pl.ds(cid * N_NODES_C + sid * 2000, 2000)])


def _aggregate(src, dst, ex, xn):
    return pl.kernel(
        _k3_body,
        out_type=(
            jax.ShapeDtypeStruct((SC_NC, N_NODES_C, HID_C), jnp.float32),
            jax.ShapeDtypeStruct((SC_NC * N_NODES_C,), jnp.float32),
        ),
        mesh=_sc_mesh(),
        scratch_types=[
            pltpu.VMEM((K3_BATCH,), jnp.int32),
            pltpu.VMEM((K3_BATCH,), jnp.int32),
            pltpu.VMEM((K3_BATCH,), jnp.int32),
            pltpu.VMEM((K3_BATCH,), jnp.int32),
            pltpu.VMEM((K3_BATCH,), jnp.float32),
            pltpu.VMEM((K3_BATCH,), jnp.float32),
            pltpu.VMEM((K3_BATCH, HID_C), jnp.float32),
            pltpu.VMEM((K3_BATCH, HID_C), jnp.float32),
            pltpu.VMEM((100, HID_C), jnp.float32),
            pltpu.VMEM((2000,), jnp.float32),
            pltpu.SemaphoreType.DMA,
            pltpu.SemaphoreType.DMA,
            pltpu.VMEM_SHARED((N_NODES_C, HID_C), jnp.float32),
            pltpu.VMEM_SHARED((N_NODES_C,), jnp.float32),
        ],
    )(src, dst, ex, xn)


def kernel(h_tokens, e_tokens, edge_index, fg_edge_index, token_emb, e_token_emb, W_ni, W_nj, W_fij, egat_bias, egat_attn, W_node, W_fwd, b_fwd, gat_W, gat_attn_l, gat_attn_r, gat_bias, W_lin, b_lin, W_cls, b_cls):
    src, dst = edge_index[0], edge_index[1]
    h = jax.nn.relu(token_emb[h_tokens])
    e = e_token_emb[e_tokens]

    w_cat2 = jnp.concatenate([W_ni, W_nj], axis=2)

    hs = (h,)
    for i in range(N_LAYERS_C):
        xi, xj = _node_matmul(hs, w_cat2[i], 2)
        g = _edge_gather_add(src, dst, xi, xj)
        (xn,) = _node_matmul(hs, W_node[i], 1)
        ef = _ef_kernel(e, W_fij[i], egat_bias[i][None, :])
        f_out, lo3d, bmax = _edge_kernel(ef, g, egat_attn[i][None, :])
        e = f_out
        ex3d = _ex_kernel(lo3d, bmax)
        hpart, hden = _aggregate(src, dst, ex3d.reshape(-1), xn)
        hd = hden.reshape(SC_NC, N_NODES_C // NODE_BLK, 1, NODE_BLK)
        hs = (hpart[0], hpart[1], hd[0], hd[1])

    h = _fwd_matmul(hs[0], hs[1], hs[2], hs[3], W_fwd, b_fwd[None, :])

    hs = jnp.sort(h, axis=-1)
    hb = hs.reshape(B_GRAPHS_C, NPG_C, HID_C)
    order = jnp.argsort(-hb[:, :, -1], axis=1)[:, :K_C]
    pooled = jnp.take_along_axis(hb, order[:, :, None], axis=1).reshape(B_GRAPHS_C, K_C * HID_C)

    ft = pooled @ gat_W
    el = jnp.sum(ft * gat_attn_l, axis=-1)
    er = jnp.sum(ft * gat_attn_r, axis=-1)
    fsrc, fdst = fg_edge_index[0], fg_edge_index[1]
    lg = jax.nn.leaky_relu(el[fsrc] + er[fdst], negative_slope=0.2)
    m = jax.ops.segment_max(lg, fdst, num_segments=B_GRAPHS_C)
    ex = jnp.exp(lg - m[fdst])
    den = jax.ops.segment_sum(ex, fdst, num_segments=B_GRAPHS_C)
    a = ex / den[fdst]
    rst = jax.ops.segment_sum(ft[fsrc] * a[:, None], fdst, num_segments=B_GRAPHS_C) + gat_bias
    h = jax.nn.relu(rst)
    h = jax.nn.relu(h @ W_lin + b_lin)
    out = h @ W_cls + b_cls
    return out.reshape(-1, 2)
